# baseline (speedup 1.0000x reference)
import jax
import jax.numpy as jnp
from jax import lax
from jax.experimental import pallas as pl
from jax.experimental.pallas import tpu as pltpu
from jax.experimental.pallas import tpu_sc as plsc

_NUM_CORES = 2
_NUM_SUBCORES = 16
_NW = _NUM_CORES * _NUM_SUBCORES

_B = 4096
_L = 200
_D = 64
_REPEATS = 2
_N = _B * _L
_PER_TILE = _N // _NW
_BATCHES_PER_TILE = _B // _NW

_GATHER_SPLITS = ((0, 128), (128, 72))
_NBUF = 2


def _body(code_hbm, table_hbm, out_hbm, idx_v, rows_v, dup_v, gsems, wsems):
    wid = lax.axis_index("s") * _NUM_CORES + lax.axis_index("c")
    tile_base = wid * _PER_TILE
    tile_batch0 = wid * _BATCHES_PER_TILE

    pltpu.sync_copy(code_hbm.at[pl.ds(tile_base, _PER_TILE)], idx_v)

    def gather_copies(step, b):
        for off, size in _GATHER_SPLITS:
            yield pltpu.make_async_copy(
                table_hbm.at[idx_v.at[pl.ds(step * _L + off, size)]],
                rows_v.at[b].at[pl.ds(off, size)],
                gsems[b],
            )

    def out_copy(step, b):
        return pltpu.make_async_copy(
            dup_v.at[b],
            out_hbm.at[tile_batch0 + step],
            wsems[b],
        )

    for b in range(_NBUF):
        for c in gather_copies(b, b):
            c.start()

    def outer(i, _):
        for b in range(_NBUF):
            step = i * _NBUF + b
            for c in gather_copies(step, b):
                c.wait()

            @pl.when(step >= _NBUF)
            def _():
                out_copy(step - _NBUF, b).wait()

            def dup(r, _):
                for d in range(_D // 16):
                    v = rows_v[b, r, pl.ds(d * 16, 16)]
                    dup_v[b, 2 * r, pl.ds(d * 16, 16)] = v
                    dup_v[b, 2 * r + 1, pl.ds(d * 16, 16)] = v
                return 0

            lax.fori_loop(0, _L, dup, 0, unroll=4)

            out_copy(step, b).start()

            @pl.when(step + _NBUF < _BATCHES_PER_TILE)
            def _():
                for c in gather_copies(step + _NBUF, b):
                    c.start()
        return 0

    lax.fori_loop(0, _BATCHES_PER_TILE // _NBUF, outer, 0)

    for b in range(_NBUF):
        out_copy(_BATCHES_PER_TILE - _NBUF + b, b).wait()


@jax.jit
def _run(code_flat, table):
    k = pl.kernel(
        _body,
        out_type=jax.ShapeDtypeStruct((_B, _L * _REPEATS, _D), jnp.float32),
        mesh=plsc.VectorSubcoreMesh(
            core_axis_name="c", subcore_axis_name="s",
            num_cores=_NUM_CORES, num_subcores=_NUM_SUBCORES,
        ),
        scratch_types=[
            pltpu.VMEM((_PER_TILE,), jnp.int32),
            pltpu.VMEM((_NBUF, _L, _D), jnp.float32),
            pltpu.VMEM((_NBUF, _L * _REPEATS, _D), jnp.float32),
            [pltpu.SemaphoreType.DMA] * _NBUF,
            [pltpu.SemaphoreType.DMA] * _NBUF,
        ],
        compiler_params=pltpu.CompilerParams(use_tc_tiling_on_sc=False),
    )
    return k(code_flat, table)


def kernel(code, table):
    code_flat = code.reshape(_N).astype(jnp.int32)
    return _run(code_flat, table)


# speedup vs baseline: 1.6007x; 1.6007x over previous
import jax
import jax.numpy as jnp
from jax import lax
from jax.experimental import pallas as pl
from jax.experimental.pallas import tpu as pltpu
from jax.experimental.pallas import tpu_sc as plsc

_NUM_CORES = 2
_NUM_SUBCORES = 16
_NW = _NUM_CORES * _NUM_SUBCORES

_B = 4096
_L = 200
_D = 64
_REPEATS = 2
_N = _B * _L
_PER_TILE = _N // _NW

_IW = 128
_STEPS = _PER_TILE // _IW
_NBUF = 2


def _body(code_hbm, table2_hbm, out_hbm, idx_v, pair_v, dup_v, gsems, wsems):
    wid = lax.axis_index("s") * _NUM_CORES + lax.axis_index("c")
    tile_base = wid * _PER_TILE

    pltpu.sync_copy(code_hbm.at[pl.ds(wid * _STEPS, _STEPS)], idx_v)

    def gather_copy(step, b):
        return pltpu.make_async_copy(
            table2_hbm.at[idx_v.at[step]],
            pair_v.at[b],
            gsems[b],
        )

    def out_copy(step, b):
        return pltpu.make_async_copy(
            dup_v.at[b],
            out_hbm.at[pl.ds(_REPEATS * (tile_base + step * _IW),
                             _REPEATS * _IW)],
            wsems[b],
        )

    for b in range(_NBUF):
        gather_copy(b, b).start()

    def outer(i, _):
        for b in range(_NBUF):
            step = i * _NBUF + b
            gather_copy(step, b).wait()

            @pl.when(step >= _NBUF)
            def _():
                out_copy(step - _NBUF, b).wait()

            def dup(r, _):
                for d in range(_D // 16):
                    v = pair_v[b, r, pl.ds(d * 16, 16)]
                    dup_v[b, 2 * r, pl.ds(d * 16, 16)] = v
                    dup_v[b, 2 * r + 1, pl.ds(d * 16, 16)] = v
                return 0

            lax.fori_loop(0, _IW, dup, 0, unroll=4)

            out_copy(step, b).start()

            @pl.when(step + _NBUF < _STEPS)
            def _():
                gather_copy(step + _NBUF, b).start()
        return 0

    lax.fori_loop(0, _STEPS // _NBUF, outer, 0)

    for b in range(_NBUF):
        out_copy(_STEPS - _NBUF + b, b).wait()


@jax.jit
def _run(code2d, table2):
    k = pl.kernel(
        _body,
        out_type=jax.ShapeDtypeStruct((_REPEATS * _N, _D), jnp.float32),
        mesh=plsc.VectorSubcoreMesh(
            core_axis_name="c", subcore_axis_name="s",
            num_cores=_NUM_CORES, num_subcores=_NUM_SUBCORES,
        ),
        scratch_types=[
            pltpu.VMEM((_STEPS, _IW), jnp.int32),
            pltpu.VMEM((_NBUF, _IW, _REPEATS * _D), jnp.float32),
            pltpu.VMEM((_NBUF, _REPEATS * _IW, _D), jnp.float32),
            [pltpu.SemaphoreType.DMA] * _NBUF,
            [pltpu.SemaphoreType.DMA] * _NBUF,
        ],
        compiler_params=pltpu.CompilerParams(use_tc_tiling_on_sc=True),
    )
    return k(code2d, table2)


def kernel(code, table):
    code2d = code.reshape(_N // _IW, _IW).astype(jnp.int32)
    table2 = jnp.concatenate([table, table], axis=1)
    out2 = _run(code2d, table2)
    return out2.reshape(_B, _L * _REPEATS, _D)
